# Initial kernel scaffold; baseline (speedup 1.0000x reference)
import jax
import jax.numpy as jnp
from jax.experimental import pallas as pl


def kernel(x):
    raise NotImplementedError("write your pallas kernel here")


# speedup vs baseline: 613.0646x; 613.0646x over previous
import jax
import jax.numpy as jnp
from jax.experimental import pallas as pl

_SIGMA_X = 1.0
_SIGMA_Y = 1.0


def _preprocess_kernel(x_ref, o_ref):
    xv = x_ref[...]
    B, _, H, W = xv.shape
    m = (xv[:, 2, :, :] > 0).astype(jnp.float32)

    ri = jax.lax.broadcasted_iota(jnp.int32, (H, H), 0)
    ci = jax.lax.broadcasted_iota(jnp.int32, (H, H), 1)
    dx = (ri - ci).astype(jnp.float32)
    kx = jnp.exp(-(dx * dx) / (2.0 * _SIGMA_X * _SIGMA_X))
    rj = jax.lax.broadcasted_iota(jnp.int32, (W, W), 0)
    cj = jax.lax.broadcasted_iota(jnp.int32, (W, W), 1)
    dy = (rj - cj).astype(jnp.float32)
    ky = jnp.exp(-(dy * dy) / (2.0 * _SIGMA_Y * _SIGMA_Y))

    count = jnp.sum(m)
    keep = count > 0.0

    o_ref[...] = xv
    for b in range(B):
        t = jnp.dot(kx, m[b], precision=jax.lax.Precision.HIGHEST)
        heat = jnp.dot(t, ky, precision=jax.lax.Precision.HIGHEST)
        mx = jnp.max(heat)
        normed = heat / jnp.where(mx == 0.0, 1.0, mx)
        o_ref[b, 2, :, :] = jnp.where(keep, normed, xv[b, 2, :, :])


@jax.jit
def kernel(x):
    return pl.pallas_call(
        _preprocess_kernel,
        out_shape=jax.ShapeDtypeStruct(x.shape, x.dtype),
    )(x)
